# Initial kernel scaffold; baseline (speedup 1.0000x reference)
import jax
import jax.numpy as jnp
from jax.experimental import pallas as pl


def kernel(x, W):
    raise NotImplementedError("write your pallas kernel here")


# speedup vs baseline: 6.8197x; 6.8197x over previous
import jax
import jax.numpy as jnp
from jax.experimental import pallas as pl

NUM_RINGS = 50
EMBED_DIM = 64
FLAT = NUM_RINGS * EMBED_DIM


def _body(x_ref, w_ref, o_ref):
    o_ref[...] = x_ref[...] + w_ref[...]


def kernel(x, W):
    B = x.shape[0]
    xf = x.reshape(B, FLAT)
    wf = W.reshape(1, FLAT)
    bm = 512
    out = pl.pallas_call(
        _body,
        grid=(B // bm,),
        in_specs=[
            pl.BlockSpec((bm, FLAT), lambda i: (i, 0)),
            pl.BlockSpec((1, FLAT), lambda i: (0, 0)),
        ],
        out_specs=pl.BlockSpec((bm, FLAT), lambda i: (i, 0)),
        out_shape=jax.ShapeDtypeStruct((B, FLAT), jnp.float32),
    )(xf, wf)
    return out.reshape(B, NUM_RINGS, EMBED_DIM)


# speedup vs baseline: 6.8367x; 1.0025x over previous
import jax
import jax.numpy as jnp
from jax.experimental import pallas as pl

NUM_RINGS = 50
EMBED_DIM = 64
FLAT = NUM_RINGS * EMBED_DIM


def _body(x_ref, w_ref, o_ref):
    o_ref[...] = x_ref[...] + w_ref[...]


def kernel(x, W):
    B = x.shape[0]
    xf = x.reshape(B, FLAT)
    wf = W.reshape(1, FLAT)
    bm = 1024
    out = pl.pallas_call(
        _body,
        grid=(B // bm,),
        in_specs=[
            pl.BlockSpec((bm, FLAT), lambda i: (i, 0)),
            pl.BlockSpec((1, FLAT), lambda i: (0, 0)),
        ],
        out_specs=pl.BlockSpec((bm, FLAT), lambda i: (i, 0)),
        out_shape=jax.ShapeDtypeStruct((B, FLAT), jnp.float32),
    )(xf, wf)
    return out.reshape(B, NUM_RINGS, EMBED_DIM)
